# baseline (speedup 1.0000x reference)
import functools

import jax
import jax.numpy as jnp
from jax import lax
from jax.experimental import pallas as pl
from jax.experimental.pallas import tpu as pltpu
from jax.experimental.pallas import tpu_sc as plsc

EMB = 64
HID = 256
B = 4096
L = 200

NC = 2
NS = 16
NW = NC * NS
BPW = B // NW
NLANE = 16
NGRP = EMB // NLANE
CHUNK = 128
LP = 256
CPR = LP // CHUNK


def _sc_body(ids_hbm, table_hbm, out_hbm, idx_v, gbuf, acc_v, sem):
    wid = lax.axis_index("s") * NC + lax.axis_index("c")
    pltpu.sync_copy(ids_hbm.at[wid], idx_v)

    def row(r, carry):
        cps = [
            pltpu.async_copy(
                table_hbm.at[idx_v.at[CPR * r + h]],
                gbuf.at[pl.ds(h * CHUNK, CHUNK)], sem)
            for h in range(CPR)
        ]
        for cp in cps:
            cp.wait()
        zero = jnp.zeros((NLANE,), jnp.float32)

        def body(t, accs):
            return tuple(accs[g] + gbuf[t, pl.ds(NLANE * g, NLANE)]
                         for g in range(NGRP))

        accs = lax.fori_loop(0, LP, body, (zero,) * NGRP, unroll=4)
        for g in range(NGRP):
            acc_v[r, pl.ds(NLANE * g, NLANE)] = accs[g]
        return carry

    lax.fori_loop(0, BPW, row, 0)
    pltpu.sync_copy(acc_v, out_hbm.at[pl.ds(wid * BPW, BPW)])


_sc_lookup = pl.kernel(
    _sc_body,
    out_type=jax.ShapeDtypeStruct((B, EMB), jnp.float32),
    mesh=plsc.VectorSubcoreMesh(core_axis_name="c", subcore_axis_name="s"),
    compiler_params=pltpu.CompilerParams(use_tc_tiling_on_sc=False),
    scratch_types=[
        pltpu.VMEM((BPW * CPR, CHUNK), jnp.int32),
        pltpu.VMEM((LP, EMB), jnp.float32),
        pltpu.VMEM((BPW, EMB), jnp.float32),
        pltpu.SemaphoreType.DMA,
    ],
)


def _mlp_body(x_ref, w1_ref, b1_ref, w2_ref, b2_ref, o_ref):
    x = x_ref[...] * (1.0 / L)
    h = jnp.dot(x, w1_ref[...], preferred_element_type=jnp.float32) + b1_ref[...]
    h = jnp.maximum(h, 0.0)
    y = jnp.dot(h, w2_ref[...], preferred_element_type=jnp.float32) + b2_ref[...]
    o_ref[...] = 1.0 / (1.0 + jnp.exp(-y))


def kernel(input_ids, emb_table, W1, b1, W2, b2):
    ids = jnp.concatenate(
        [input_ids.astype(jnp.int32),
         jnp.zeros((B, LP - L), jnp.int32)], axis=1)
    ids = ids.reshape(NW, BPW * CPR, CHUNK)
    sums = _sc_lookup(ids, emb_table)
    return pl.pallas_call(
        _mlp_body,
        out_shape=jax.ShapeDtypeStruct((B, 1), jnp.float32),
    )(sums, W1, b1.reshape(1, HID), W2, b2.reshape(1, 1))
